# baseline (speedup 1.0000x reference)
import functools

import jax
import jax.numpy as jnp
from jax.experimental import pallas as pl
from jax.experimental.pallas import tpu as pltpu

NN = 10000
HID = 256
T_EDGE = 512
B_NODE = 1000


def _silu(x):
    return x * jax.nn.sigmoid(x)


def _edge_body(pre, crow, ccol, ea, wrad, wea, b0, w1, b1,
               wc0, bc0, wc1, wx0, bx0, wx1, ef_o, tr_o):
    cr = crow[...]
    cc_ = ccol[...]
    d = cr - cc_
    radial = jnp.sum(d * d, axis=1, keepdims=True)
    norm = jnp.sqrt(radial + 1e-8)
    dn = d / (norm + 1.0)
    a1 = cr[:, 1:2]; a2 = cr[:, 2:3]; a0 = cr[:, 0:1]
    b1_ = cc_[:, 1:2]; b2 = cc_[:, 2:3]; b0_ = cc_[:, 0:1]
    c0 = a1 * b2 - a2 * b1_
    c1 = a2 * b0_ - a0 * b2
    c2 = a0 * b1_ - a1 * b0_
    lane = jax.lax.broadcasted_iota(jnp.int32, d.shape, 1)
    cx = jnp.where(lane == 0, c0, jnp.where(lane == 1, c1,
                   jnp.where(lane == 2, c2, 0.0)))
    nrm = jnp.sqrt(jnp.sum(cx * cx, axis=1, keepdims=True) + 1e-8)
    cxn = cx / (nrm + 1.0)

    z = pre[...] + radial * wrad[...] + jnp.dot(
        ea[...], wea[...], preferred_element_type=jnp.float32, precision=jax.lax.Precision.HIGHEST) + b0[...]
    t0 = _silu(z)
    ef = _silu(jnp.dot(t0, w1[...], preferred_element_type=jnp.float32, precision=jax.lax.Precision.HIGHEST)
               + b1[...])
    g0 = _silu(jnp.dot(ef, wc0[...], preferred_element_type=jnp.float32, precision=jax.lax.Precision.HIGHEST)
               + bc0[...])
    cm = jnp.sum(g0 * wc1[...], axis=1, keepdims=True)
    g1 = _silu(jnp.dot(ef, wx0[...], preferred_element_type=jnp.float32, precision=jax.lax.Precision.HIGHEST)
               + bx0[...])
    cxm = jnp.sum(g1 * wx1[...], axis=1, keepdims=True)
    ef_o[...] = ef
    tr_o[...] = dn * cm + cxn * cxm


def _edge_call(pre, crow, ccol, ea, wrad, wea, b0, w1, b1,
               wc0, bc0, wc1, wx0, bx0, wx1):
    E = pre.shape[0]
    grid = E // T_EDGE
    edge_spec = lambda w: pl.BlockSpec((T_EDGE, w), lambda i: (i, 0))
    const_spec = lambda s: pl.BlockSpec(s, lambda i: (0, 0))
    in_specs = [
        edge_spec(HID), edge_spec(16), edge_spec(16), edge_spec(16),
        const_spec((1, HID)), const_spec((16, HID)), const_spec((1, HID)),
        const_spec((HID, HID)), const_spec((1, HID)),
        const_spec((HID, HID)), const_spec((1, HID)), const_spec((1, HID)),
        const_spec((HID, HID)), const_spec((1, HID)), const_spec((1, HID)),
    ]
    out_specs = [edge_spec(HID), edge_spec(16)]
    return pl.pallas_call(
        _edge_body,
        grid=(grid,),
        in_specs=in_specs,
        out_specs=out_specs,
        out_shape=[jax.ShapeDtypeStruct((E, HID), jnp.float32),
                   jax.ShapeDtypeStruct((E, 16), jnp.float32)],
    )(pre, crow, ccol, ea, wrad, wea, b0, w1, b1,
      wc0, bc0, wc1, wx0, bx0, wx1)


def _emb_body(h, we, be, wr, wc, hh_o, tr_o, tc_o):
    hh = jnp.dot(h[...], we[...], preferred_element_type=jnp.float32, precision=jax.lax.Precision.HIGHEST) + be[...]
    hh_o[...] = hh
    tr_o[...] = jnp.dot(hh, wr[...], preferred_element_type=jnp.float32, precision=jax.lax.Precision.HIGHEST)
    tc_o[...] = jnp.dot(hh, wc[...], preferred_element_type=jnp.float32, precision=jax.lax.Precision.HIGHEST)


def _emb_call(h, we, be, wr, wc):
    n, in_nf = h.shape
    grid = n // B_NODE
    const_spec = lambda s: pl.BlockSpec(s, lambda i: (0, 0))
    row_spec = lambda w: pl.BlockSpec((B_NODE, w), lambda i: (i, 0))
    return pl.pallas_call(
        _emb_body,
        grid=(grid,),
        in_specs=[row_spec(in_nf), const_spec((in_nf, HID)),
                  const_spec((1, HID)), const_spec((HID, HID)),
                  const_spec((HID, HID))],
        out_specs=[row_spec(HID), row_spec(HID), row_spec(HID)],
        out_shape=[jax.ShapeDtypeStruct((n, HID), jnp.float32)] * 3,
    )(h, we, be, wr, wc)


def _node_body(h, aggh, wn0h, wn0a, bn0, wn1, bn1, wr, wc,
               hn_o, tr_o, tc_o):
    m = _silu(jnp.dot(h[...], wn0h[...], preferred_element_type=jnp.float32, precision=jax.lax.Precision.HIGHEST)
              + jnp.dot(aggh[...], wn0a[...],
                        preferred_element_type=jnp.float32, precision=jax.lax.Precision.HIGHEST) + bn0[...])
    hn = h[...] + jnp.dot(m, wn1[...],
                          preferred_element_type=jnp.float32, precision=jax.lax.Precision.HIGHEST) + bn1[...]
    hn_o[...] = hn
    tr_o[...] = jnp.dot(hn, wr[...], preferred_element_type=jnp.float32, precision=jax.lax.Precision.HIGHEST)
    tc_o[...] = jnp.dot(hn, wc[...], preferred_element_type=jnp.float32, precision=jax.lax.Precision.HIGHEST)


def _node_call(h, aggh, wn0h, wn0a, bn0, wn1, bn1, wr, wc):
    n = h.shape[0]
    grid = n // B_NODE
    const_spec = lambda s: pl.BlockSpec(s, lambda i: (0, 0))
    row_spec = lambda w: pl.BlockSpec((B_NODE, w), lambda i: (i, 0))
    return pl.pallas_call(
        _node_body,
        grid=(grid,),
        in_specs=[row_spec(HID), row_spec(HID),
                  const_spec((HID, HID)), const_spec((HID, HID)),
                  const_spec((1, HID)), const_spec((HID, HID)),
                  const_spec((1, HID)), const_spec((HID, HID)),
                  const_spec((HID, HID))],
        out_specs=[row_spec(HID)] * 3,
        out_shape=[jax.ShapeDtypeStruct((n, HID), jnp.float32)] * 3,
    )(h, aggh, wn0h, wn0a, bn0, wn1, bn1, wr, wc)


def _node_last_body(h, aggh, wn0h, wn0a, bn0, wn1, bn1, wo, bo, ho_o):
    m = _silu(jnp.dot(h[...], wn0h[...], preferred_element_type=jnp.float32, precision=jax.lax.Precision.HIGHEST)
              + jnp.dot(aggh[...], wn0a[...],
                        preferred_element_type=jnp.float32, precision=jax.lax.Precision.HIGHEST) + bn0[...])
    hn = h[...] + jnp.dot(m, wn1[...],
                          preferred_element_type=jnp.float32, precision=jax.lax.Precision.HIGHEST) + bn1[...]
    ho_o[...] = jnp.dot(hn, wo[...],
                        preferred_element_type=jnp.float32, precision=jax.lax.Precision.HIGHEST) + bo[...]


def _node_last_call(h, aggh, wn0h, wn0a, bn0, wn1, bn1, wo, bo):
    n = h.shape[0]
    out_nf = wo.shape[1]
    grid = n // B_NODE
    const_spec = lambda s: pl.BlockSpec(s, lambda i: (0, 0))
    row_spec = lambda w: pl.BlockSpec((B_NODE, w), lambda i: (i, 0))
    return pl.pallas_call(
        _node_last_body,
        grid=(grid,),
        in_specs=[row_spec(HID), row_spec(HID),
                  const_spec((HID, HID)), const_spec((HID, HID)),
                  const_spec((1, HID)), const_spec((HID, HID)),
                  const_spec((1, HID)), const_spec((HID, out_nf)),
                  const_spec((1, out_nf))],
        out_specs=[row_spec(out_nf)],
        out_shape=[jax.ShapeDtypeStruct((n, out_nf), jnp.float32)],
    )(h, aggh, wn0h, wn0a, bn0, wn1, bn1, wo, bo)[0]


def kernel(h, x, edges, edge_attr, params):
    n = h.shape[0]
    row, col = edges[0], edges[1]
    coordp = jnp.pad(x, ((0, 0), (0, 13)))

    gcl = params["gcl"]
    nl = len(gcl)

    def edge_w(lp):
        w0 = lp["edge0"]["W"]
        return (w0[:HID], w0[HID:2 * HID], w0[2 * HID:2 * HID + 1],
                w0[2 * HID + 1:], lp["edge0"]["b"][None, :])

    wr0, wc0_, _, _, _ = edge_w(gcl[0])
    hh, tr, tc = _emb_call(h, params["emb"]["W"],
                           params["emb"]["b"][None, :], wr0, wc0_)

    for l in range(nl):
        lp = gcl[l]
        _, _, wrad, wea, b0 = edge_w(lp)
        pre = tr[row] + tc[col]
        crow = coordp[row]
        ccol = coordp[col]
        ef, trans = _edge_call(
            pre, crow, ccol, edge_attr, wrad, wea, b0,
            lp["edge1"]["W"], lp["edge1"]["b"][None, :],
            lp["coord0"]["W"], lp["coord0"]["b"][None, :],
            lp["coord1"]["W"].T,
            lp["cross0"]["W"], lp["cross0"]["b"][None, :],
            lp["cross1"]["W"].T)
        aggh = jax.ops.segment_sum(ef, row, num_segments=n)
        coordp = coordp + jax.ops.segment_sum(trans, row, num_segments=n)
        nw0 = lp["node0"]["W"]
        if l + 1 < nl:
            wrn, wcn, _, _, _ = edge_w(gcl[l + 1])
            hh, tr, tc = _node_call(
                hh, aggh, nw0[:HID], nw0[HID:], lp["node0"]["b"][None, :],
                lp["node1"]["W"], lp["node1"]["b"][None, :], wrn, wcn)
        else:
            h_out = _node_last_call(
                hh, aggh, nw0[:HID], nw0[HID:], lp["node0"]["b"][None, :],
                lp["node1"]["W"], lp["node1"]["b"][None, :],
                params["emb_out"]["W"], params["emb_out"]["b"][None, :])
    return (h_out, coordp[:, :3])


# speedup vs baseline: 1.4244x; 1.4244x over previous
import functools

import jax
import jax.numpy as jnp
from jax import lax
from jax.experimental import pallas as pl
from jax.experimental.pallas import tpu as pltpu
from jax.experimental.pallas import tpu_sc as plsc

NN = 10000
HID = 256
T_EDGE = 512
B_NODE = 1000
SC_NC = 2
SC_NS = 16
SC_NW = SC_NC * SC_NS
GK = 200


def _sc_gather_call(trp, tcp, row, col):
    e = row.shape[0]
    d = trp.shape[1]
    per_w = e // SC_NW
    chunks = per_w // GK
    mesh = plsc.VectorSubcoreMesh(core_axis_name="c", subcore_axis_name="s")

    @functools.partial(
        pl.kernel, mesh=mesh,
        out_type=[jax.ShapeDtypeStruct((e, d), jnp.float32),
                  jax.ShapeDtypeStruct((e, d), jnp.float32)],
        scratch_types=[pltpu.VMEM((GK,), jnp.int32),
                       pltpu.VMEM((GK,), jnp.int32),
                       pltpu.VMEM((GK, d), jnp.float32),
                       pltpu.VMEM((GK, d), jnp.float32),
                       pltpu.SemaphoreType.DMA],
    )
    def k(trp_h, tcp_h, row_h, col_h, gr_h, gc_h,
          idx_r, idx_c, buf_r, buf_c, sem):
        wid = lax.axis_index("s") * SC_NC + lax.axis_index("c")
        base = wid * per_w

        def body(g, carry):
            off = base + g * GK
            pltpu.sync_copy(row_h.at[pl.ds(off, GK)], idx_r)
            pltpu.sync_copy(col_h.at[pl.ds(off, GK)], idx_c)
            cp1 = pltpu.make_async_copy(trp_h.at[idx_r], buf_r, sem)
            cp2 = pltpu.make_async_copy(tcp_h.at[idx_c], buf_c, sem)
            cp1.start()
            cp2.start()
            cp1.wait()
            cp2.wait()
            pltpu.sync_copy(buf_r, gr_h.at[pl.ds(off, GK)])
            pltpu.sync_copy(buf_c, gc_h.at[pl.ds(off, GK)])
            return carry

        lax.fori_loop(0, chunks, body, 0)

    return k(trp, tcp, row, col)


def _bdot(x, w):
    return jnp.dot(x.astype(jnp.bfloat16), w.astype(jnp.bfloat16),
                   preferred_element_type=jnp.float32)


def _btrunc(x):
    return x.astype(jnp.bfloat16).astype(jnp.float32)

def _silu(x):
    return x * jax.nn.sigmoid(x)


def _edge_body(gr, gc, crow, ccol, ea, wrad, wea, b0, w1, b1,
               wc0, bc0, wc1, wx0, bx0, wx1, ef_o, tr_o):
    cr = crow[...]
    cc_ = ccol[...]
    pre = gr[...] + gc[...]
    d = cr - cc_
    radial = jnp.sum(d * d, axis=1, keepdims=True)
    norm = jnp.sqrt(radial + 1e-8)
    dn = d / (norm + 1.0)
    a1 = cr[:, 1:2]; a2 = cr[:, 2:3]; a0 = cr[:, 0:1]
    b1_ = cc_[:, 1:2]; b2 = cc_[:, 2:3]; b0_ = cc_[:, 0:1]
    c0 = a1 * b2 - a2 * b1_
    c1 = a2 * b0_ - a0 * b2
    c2 = a0 * b1_ - a1 * b0_
    lane = jax.lax.broadcasted_iota(jnp.int32, d.shape, 1)
    cx = jnp.where(lane == 0, c0, jnp.where(lane == 1, c1,
                   jnp.where(lane == 2, c2, 0.0)))
    nrm = jnp.sqrt(jnp.sum(cx * cx, axis=1, keepdims=True) + 1e-8)
    cxn = cx / (nrm + 1.0)

    z = pre + _btrunc(radial) * _btrunc(wrad[...]) + _bdot(
        ea[...], wea[...]) + b0[...]
    t0 = _silu(z)
    ef = _silu(_bdot(t0, w1[...]) + b1[...])
    g0 = _silu(_bdot(ef, wc0[...]) + bc0[...])
    cm = jnp.sum(_btrunc(g0) * _btrunc(wc1[...]), axis=1, keepdims=True)
    g1 = _silu(_bdot(ef, wx0[...]) + bx0[...])
    cxm = jnp.sum(_btrunc(g1) * _btrunc(wx1[...]), axis=1, keepdims=True)
    ef_o[...] = ef
    tr_o[...] = dn * cm + cxn * cxm


def _edge_call(gr, gc, crow, ccol, ea, wrad, wea, b0, w1, b1,
               wc0, bc0, wc1, wx0, bx0, wx1):
    E = gr.shape[0]
    grid = E // T_EDGE
    edge_spec = lambda w: pl.BlockSpec((T_EDGE, w), lambda i: (i, 0))
    const_spec = lambda s: pl.BlockSpec(s, lambda i: (0, 0))
    in_specs = [
        edge_spec(HID), edge_spec(HID), edge_spec(16), edge_spec(16),
        edge_spec(16),
        const_spec((1, HID)), const_spec((16, HID)), const_spec((1, HID)),
        const_spec((HID, HID)), const_spec((1, HID)),
        const_spec((HID, HID)), const_spec((1, HID)), const_spec((1, HID)),
        const_spec((HID, HID)), const_spec((1, HID)), const_spec((1, HID)),
    ]
    out_specs = [edge_spec(HID), edge_spec(16)]
    return pl.pallas_call(
        _edge_body,
        grid=(grid,),
        in_specs=in_specs,
        out_specs=out_specs,
        out_shape=[jax.ShapeDtypeStruct((E, HID), jnp.float32),
                   jax.ShapeDtypeStruct((E, 16), jnp.float32)],
    )(gr, gc, crow, ccol, ea, wrad, wea, b0, w1, b1,
      wc0, bc0, wc1, wx0, bx0, wx1)


def _emb_body(h, we, be, wr, wc, hh_o, tr_o, tc_o):
    hh = _bdot(h[...], we[...]) + be[...]
    hh_o[...] = hh
    tr_o[...] = _bdot(hh, wr[...])
    tc_o[...] = _bdot(hh, wc[...])


def _emb_call(h, we, be, wr, wc):
    n, in_nf = h.shape
    grid = n // B_NODE
    const_spec = lambda s: pl.BlockSpec(s, lambda i: (0, 0))
    row_spec = lambda w: pl.BlockSpec((B_NODE, w), lambda i: (i, 0))
    return pl.pallas_call(
        _emb_body,
        grid=(grid,),
        in_specs=[row_spec(in_nf), const_spec((in_nf, HID)),
                  const_spec((1, HID)), const_spec((HID, HID)),
                  const_spec((HID, HID))],
        out_specs=[row_spec(HID), row_spec(HID), row_spec(HID)],
        out_shape=[jax.ShapeDtypeStruct((n, HID), jnp.float32)] * 3,
    )(h, we, be, wr, wc)


def _node_body(h, aggh, wn0h, wn0a, bn0, wn1, bn1, wr, wc,
               hn_o, tr_o, tc_o):
    m = _silu(_bdot(h[...], wn0h[...]) + _bdot(aggh[...], wn0a[...])
              + bn0[...])
    hn = h[...] + _bdot(m, wn1[...]) + bn1[...]
    hn_o[...] = hn
    tr_o[...] = _bdot(hn, wr[...])
    tc_o[...] = _bdot(hn, wc[...])


def _node_call(h, aggh, wn0h, wn0a, bn0, wn1, bn1, wr, wc):
    n = h.shape[0]
    grid = n // B_NODE
    const_spec = lambda s: pl.BlockSpec(s, lambda i: (0, 0))
    row_spec = lambda w: pl.BlockSpec((B_NODE, w), lambda i: (i, 0))
    return pl.pallas_call(
        _node_body,
        grid=(grid,),
        in_specs=[row_spec(HID), row_spec(HID),
                  const_spec((HID, HID)), const_spec((HID, HID)),
                  const_spec((1, HID)), const_spec((HID, HID)),
                  const_spec((1, HID)), const_spec((HID, HID)),
                  const_spec((HID, HID))],
        out_specs=[row_spec(HID)] * 3,
        out_shape=[jax.ShapeDtypeStruct((n, HID), jnp.float32)] * 3,
    )(h, aggh, wn0h, wn0a, bn0, wn1, bn1, wr, wc)


def _node_last_body(h, aggh, wn0h, wn0a, bn0, wn1, bn1, wo, bo, ho_o):
    m = _silu(_bdot(h[...], wn0h[...]) + _bdot(aggh[...], wn0a[...])
              + bn0[...])
    hn = h[...] + _bdot(m, wn1[...]) + bn1[...]
    ho_o[...] = _bdot(hn, wo[...]) + bo[...]


def _node_last_call(h, aggh, wn0h, wn0a, bn0, wn1, bn1, wo, bo):
    n = h.shape[0]
    out_nf = wo.shape[1]
    grid = n // B_NODE
    const_spec = lambda s: pl.BlockSpec(s, lambda i: (0, 0))
    row_spec = lambda w: pl.BlockSpec((B_NODE, w), lambda i: (i, 0))
    return pl.pallas_call(
        _node_last_body,
        grid=(grid,),
        in_specs=[row_spec(HID), row_spec(HID),
                  const_spec((HID, HID)), const_spec((HID, HID)),
                  const_spec((1, HID)), const_spec((HID, HID)),
                  const_spec((1, HID)), const_spec((HID, out_nf)),
                  const_spec((1, out_nf))],
        out_specs=[row_spec(out_nf)],
        out_shape=[jax.ShapeDtypeStruct((n, out_nf), jnp.float32)],
    )(h, aggh, wn0h, wn0a, bn0, wn1, bn1, wo, bo)[0]


def kernel(h, x, edges, edge_attr, params):
    n = h.shape[0]
    row, col = edges[0], edges[1]
    coordp = jnp.pad(x, ((0, 0), (0, 13)))

    gcl = params["gcl"]
    nl = len(gcl)

    def edge_w(lp):
        w0 = lp["edge0"]["W"]
        return (w0[:HID], w0[HID:2 * HID], w0[2 * HID:2 * HID + 1],
                w0[2 * HID + 1:], lp["edge0"]["b"][None, :])

    wr0, wc0_, _, _, _ = edge_w(gcl[0])
    hh, tr, tc = _emb_call(h, params["emb"]["W"],
                           params["emb"]["b"][None, :], wr0, wc0_)

    for l in range(nl):
        lp = gcl[l]
        _, _, wrad, wea, b0 = edge_w(lp)
        gr, gc = _sc_gather_call(tr, tc, row, col)
        crow = coordp[row]
        ccol = coordp[col]
        ef, trans = _edge_call(
            gr, gc, crow, ccol, edge_attr, wrad, wea, b0,
            lp["edge1"]["W"], lp["edge1"]["b"][None, :],
            lp["coord0"]["W"], lp["coord0"]["b"][None, :],
            lp["coord1"]["W"].T,
            lp["cross0"]["W"], lp["cross0"]["b"][None, :],
            lp["cross1"]["W"].T)
        aggh = jax.ops.segment_sum(ef, row, num_segments=n)
        coordp = coordp + jax.ops.segment_sum(trans, row, num_segments=n)
        nw0 = lp["node0"]["W"]
        if l + 1 < nl:
            wrn, wcn, _, _, _ = edge_w(gcl[l + 1])
            hh, tr, tc = _node_call(
                hh, aggh, nw0[:HID], nw0[HID:], lp["node0"]["b"][None, :],
                lp["node1"]["W"], lp["node1"]["b"][None, :], wrn, wcn)
        else:
            h_out = _node_last_call(
                hh, aggh, nw0[:HID], nw0[HID:], lp["node0"]["b"][None, :],
                lp["node1"]["W"], lp["node1"]["b"][None, :],
                params["emb_out"]["W"], params["emb_out"]["b"][None, :])
    return (h_out, coordp[:, :3])
